# baseline (speedup 1.0000x reference)
import functools

import jax
import jax.numpy as jnp
from jax import lax
from jax.experimental import pallas as pl
from jax.experimental.pallas import tpu as pltpu
from jax.experimental.pallas import tpu_sc as plsc

N = 10000
E = 160000
D_IN = 256
D_OUT = 78
H = 256
H_DEC = 128

NC = 2
NS = 16
NW = NC * NS


BN = 2000


def _tables_body(x_ref, ws_ref, wd_ref, b_ref, ts_ref, td_ref):
    x = x_ref[...]
    ts_ref[...] = jnp.dot(
        x, ws_ref[...], preferred_element_type=jnp.float32
    ).astype(jnp.bfloat16)
    td_ref[...] = (jnp.dot(x, wd_ref[...], preferred_element_type=jnp.float32)
                   + b_ref[...]).astype(jnp.bfloat16)


_tables = pl.pallas_call(
    _tables_body,
    grid=(N // BN,),
    in_specs=[
        pl.BlockSpec((BN, D_IN), lambda i: (i, 0)),
        pl.BlockSpec((D_IN, H), lambda i: (0, 0)),
        pl.BlockSpec((D_IN, H), lambda i: (0, 0)),
        pl.BlockSpec((1, H), lambda i: (0, 0)),
    ],
    out_specs=[
        pl.BlockSpec((BN, H), lambda i: (i, 0)),
        pl.BlockSpec((BN, H), lambda i: (i, 0)),
    ],
    out_shape=[
        jax.ShapeDtypeStruct((N, H), jnp.bfloat16),
        jax.ShapeDtypeStruct((N, H), jnp.bfloat16),
    ],
)


EW = E // NW
KG = 40
NIT = EW // KG
HP = H // 2


@functools.partial(
    pl.kernel,
    out_type=[
        jax.ShapeDtypeStruct((E, HP), jnp.int32),
        jax.ShapeDtypeStruct((E, HP), jnp.int32),
    ],
    mesh=plsc.VectorSubcoreMesh(core_axis_name="c", subcore_axis_name="s"),
    scratch_types=[
        pltpu.VMEM((KG,), jnp.int32),
        pltpu.VMEM((KG,), jnp.int32),
        pltpu.VMEM((KG, HP), jnp.int32),
        pltpu.VMEM((KG, HP), jnp.int32),
        pltpu.SemaphoreType.DMA,
        pltpu.SemaphoreType.DMA,
    ],
    compiler_params=pltpu.CompilerParams(needs_layout_passes=False),
)
def _gather_sc(src_hbm, dst_hbm, ts_hbm, td_hbm, gs_hbm, gd_hbm,
               idxs_v, idxd_v, bs_v, bd_v, sem1, sem2):
    wid = lax.axis_index("s") * NC + lax.axis_index("c")
    base = wid * EW

    def body(i, carry):
        b = base + i * KG
        pltpu.sync_copy(src_hbm.at[pl.ds(b, KG)], idxs_v)
        pltpu.sync_copy(dst_hbm.at[pl.ds(b, KG)], idxd_v)
        cp1 = pltpu.async_copy(ts_hbm.at[idxs_v], bs_v, sem1)
        cp2 = pltpu.async_copy(td_hbm.at[idxd_v], bd_v, sem2)
        cp1.wait()
        cp2.wait()
        pltpu.sync_copy(bs_v, gs_hbm.at[pl.ds(b, KG)])
        pltpu.sync_copy(bd_v, gd_hbm.at[pl.ds(b, KG)])
        return carry

    lax.fori_loop(0, NIT, body, 0)


BE = 2000


def _edges_body(attr_ref, gs_ref, gd_ref, dst_ref,
                ee_w0_ref, ee_b0_ref, ee_w1_ref, ee_b1_ref, ee_w2_ref,
                ee_b2_ref, ee_g_ref, ee_bb_ref,
                we_ref, pe_w1_ref, pe_b1_ref, pe_w2_ref,
                pe_b2_ref, pe_g_ref, pe_bb_ref,
                e2_ref, idx2_ref):
    def silu(v):
        return v * jax.nn.sigmoid(v)

    a = attr_ref[...]
    w0 = ee_w0_ref[...]
    h = silu(a[:, 0:1] * w0[0:1, :] + a[:, 1:2] * w0[1:2, :] + ee_b0_ref[...])
    h = silu(jnp.dot(h, ee_w1_ref[...], preferred_element_type=jnp.float32)
             + ee_b1_ref[...])
    epre = (jnp.dot(h, ee_w2_ref[...], preferred_element_type=jnp.float32)
            + ee_b2_ref[...])
    m = (epre[:, 0:1] + epre[:, 1:2]) * 0.5
    d0 = epre[:, 0:1] - m
    r = lax.rsqrt(d0 * d0 + 1e-5)
    g = ee_g_ref[...]
    bb = ee_bb_ref[...]
    e0 = d0 * r * g[:, 0:1] + bb[:, 0:1]
    e1 = -d0 * r * g[:, 1:2] + bb[:, 1:2]

    we = we_ref[...]
    h2 = silu(e0 * we[0:1, :] + e1 * we[1:2, :]
              + gs_ref[...].astype(jnp.float32)
              + gd_ref[...].astype(jnp.float32))
    h2 = silu(jnp.dot(h2, pe_w1_ref[...], preferred_element_type=jnp.float32)
              + pe_b1_ref[...])
    q = (jnp.dot(h2, pe_w2_ref[...], preferred_element_type=jnp.float32)
         + pe_b2_ref[...])
    m2 = (q[:, 0:1] + q[:, 1:2]) * 0.5
    dq = q[:, 0:1] - m2
    r2 = lax.rsqrt(dq * dq + 1e-5)
    g2 = pe_g_ref[...]
    bb2 = pe_bb_ref[...]
    e2_0 = dq * r2 * g2[:, 0:1] + bb2[:, 0:1] + e0
    e2_1 = -dq * r2 * g2[:, 1:2] + bb2[:, 1:2] + e1
    e2_ref[...] = jnp.concatenate([e2_0, e2_1], axis=1)

    d = dst_ref[...]
    idx2_ref[...] = 2 * d + lax.broadcasted_iota(jnp.int32, (BE, 2), 1)


def _w_spec(shape):
    return pl.BlockSpec(shape, lambda i: tuple(0 for _ in shape))


_edges = pl.pallas_call(
    _edges_body,
    grid=(E // BE,),
    in_specs=[
        pl.BlockSpec((BE, 2), lambda i: (i, 0)),
        pl.BlockSpec((BE, H), lambda i: (i, 0)),
        pl.BlockSpec((BE, H), lambda i: (i, 0)),
        pl.BlockSpec((BE, 1), lambda i: (i, 0)),
        _w_spec((2, H)), _w_spec((1, H)), _w_spec((H, H)), _w_spec((1, H)),
        _w_spec((H, 2)), _w_spec((1, 2)), _w_spec((1, 2)), _w_spec((1, 2)),
        _w_spec((2, H)), _w_spec((H, H)), _w_spec((1, H)),
        _w_spec((H, 2)), _w_spec((1, 2)), _w_spec((1, 2)), _w_spec((1, 2)),
    ],
    out_specs=[
        pl.BlockSpec((BE, 2), lambda i: (i, 0)),
        pl.BlockSpec((BE, 2), lambda i: (i, 0)),
    ],
    out_shape=[
        jax.ShapeDtypeStruct((E, 2), jnp.float32),
        jax.ShapeDtypeStruct((E, 2), jnp.int32),
    ],
)


TSC = 8
CH = 2 * E // TSC
SUB = 2000
NSUB = CH // SUB


@functools.partial(
    pl.kernel,
    out_type=jax.ShapeDtypeStruct((TSC, 2 * N), jnp.float32),
    mesh=plsc.VectorSubcoreMesh(core_axis_name="c", subcore_axis_name="s"),
    scratch_types=[
        pltpu.VMEM((SUB,), jnp.int32),
        pltpu.VMEM((SUB,), jnp.float32),
        pltpu.VMEM((2 * N,), jnp.float32),
    ],
    compiler_params=pltpu.CompilerParams(needs_layout_passes=False),
)
def _scatter_sc(idx_hbm, val_hbm, out_hbm, idx_v, val_v, acc_v):
    wid = lax.axis_index("s") * NC + lax.axis_index("c")

    @pl.when(wid < TSC)
    def _():
        def zero(i, carry):
            acc_v[pl.ds(i * 16, 16)] = jnp.zeros((16,), jnp.float32)
            return carry

        lax.fori_loop(0, (2 * N) // 16, zero, 0)

        def sub(s, carry):
            b = wid * CH + s * SUB
            pltpu.sync_copy(idx_hbm.at[pl.ds(b, SUB)], idx_v)
            pltpu.sync_copy(val_hbm.at[pl.ds(b, SUB)], val_v)

            def inner(j, c2):
                iv = idx_v[pl.ds(j * 16, 16)]
                vv = val_v[pl.ds(j * 16, 16)]
                plsc.addupdate_scatter(acc_v, [iv], vv)
                return c2

            lax.fori_loop(0, SUB // 16, inner, 0)
            return carry

        lax.fori_loop(0, NSUB, sub, 0)
        pltpu.sync_copy(acc_v, out_hbm.at[wid])


def _nodes_body(x_ref, agg_ref, st_ref,
                wx_ref, wa_ref, pn_b0_ref, pn_w1_ref, pn_b1_ref,
                pn_w2_ref, pn_b2_ref, pn_g_ref, pn_bb_ref,
                nd_w0_ref, nd_b0_ref, nd_w1_ref, nd_b1_ref,
                nd_w2_ref, nd_b2_ref, out_ref):
    def silu(v):
        return v * jax.nn.sigmoid(v)

    agg = agg_ref[0]
    for k in range(1, TSC):
        agg = agg + agg_ref[k]
    wa = wa_ref[...]
    x = x_ref[...]
    h = silu(jnp.dot(x, wx_ref[...], preferred_element_type=jnp.float32)
             + agg[:, 0:1] * wa[0:1, :] + agg[:, 1:2] * wa[1:2, :]
             + pn_b0_ref[...])
    h = silu(jnp.dot(h, pn_w1_ref[...], preferred_element_type=jnp.float32)
             + pn_b1_ref[...])
    xp = (jnp.dot(h, pn_w2_ref[...], preferred_element_type=jnp.float32)
          + pn_b2_ref[...])
    mu = jnp.mean(xp, axis=-1, keepdims=True)
    ctr = xp - mu
    va = jnp.mean(ctr * ctr, axis=-1, keepdims=True)
    x2 = ctr * lax.rsqrt(va + 1e-5) * pn_g_ref[...] + pn_bb_ref[...] + x
    dd = silu(jnp.dot(x2, nd_w0_ref[...], preferred_element_type=jnp.float32)
              + nd_b0_ref[...])
    dd = silu(jnp.dot(dd, nd_w1_ref[...], preferred_element_type=jnp.float32)
              + nd_b1_ref[...])
    out_ref[...] = (jnp.dot(dd, nd_w2_ref[...],
                            preferred_element_type=jnp.float32)
                    + nd_b2_ref[...] + st_ref[...])


_nodes = pl.pallas_call(
    _nodes_body,
    grid=(N // BN,),
    in_specs=[
        pl.BlockSpec((BN, D_IN), lambda i: (i, 0)),
        pl.BlockSpec((TSC, BN, 2), lambda i: (0, i, 0)),
        pl.BlockSpec((BN, D_OUT), lambda i: (i, 0)),
        _w_spec((D_IN, H)), _w_spec((2, H)), _w_spec((1, H)),
        _w_spec((H, H)), _w_spec((1, H)),
        _w_spec((H, D_IN)), _w_spec((1, D_IN)), _w_spec((1, D_IN)),
        _w_spec((1, D_IN)),
        _w_spec((D_IN, H_DEC)), _w_spec((1, H_DEC)),
        _w_spec((H_DEC, H_DEC)), _w_spec((1, H_DEC)),
        _w_spec((H_DEC, D_OUT)), _w_spec((1, D_OUT)),
    ],
    out_specs=pl.BlockSpec((BN, D_OUT), lambda i: (i, 0)),
    out_shape=jax.ShapeDtypeStruct((N, D_OUT), jnp.float32),
)


def kernel(processor_features, start_features, edge_attr, edge_index,
           ee_w0, ee_b0, ee_w1, ee_b1, ee_w2, ee_b2, ee_ln_g, ee_ln_b,
           pe_w0, pe_b0, pe_w1, pe_b1, pe_w2, pe_b2, pe_ln_g, pe_ln_b,
           pn_w0, pn_b0, pn_w1, pn_b1, pn_w2, pn_b2, pn_ln_g, pn_ln_b,
           nd_w0, nd_b0, nd_w1, nd_b1, nd_w2, nd_b2):
    x = processor_features
    src = edge_index[0]
    dst = edge_index[1]
    we = pe_w0[0:2]
    ws = pe_w0[2:2 + D_IN]
    wd = pe_w0[2 + D_IN:2 + 2 * D_IN]

    ts, td = _tables(x, ws, wd, pe_b0.reshape(1, H))
    ts_p = lax.bitcast_convert_type(ts.reshape(N, H // 2, 2), jnp.int32)
    td_p = lax.bitcast_convert_type(td.reshape(N, H // 2, 2), jnp.int32)
    gs_p, gd_p = _gather_sc(src, dst, ts_p, td_p)
    gs = lax.bitcast_convert_type(gs_p, jnp.bfloat16).reshape(E, H)
    gd = lax.bitcast_convert_type(gd_p, jnp.bfloat16).reshape(E, H)
    e2, idx2 = _edges(
        edge_attr, gs, gd, dst.reshape(E, 1),
        ee_w0, ee_b0.reshape(1, H), ee_w1, ee_b1.reshape(1, H),
        ee_w2, ee_b2.reshape(1, 2), ee_ln_g.reshape(1, 2),
        ee_ln_b.reshape(1, 2),
        we, pe_w1, pe_b1.reshape(1, H), pe_w2, pe_b2.reshape(1, 2),
        pe_ln_g.reshape(1, 2), pe_ln_b.reshape(1, 2))
    partials = _scatter_sc(idx2.reshape(2 * E), e2.reshape(2 * E))
    aggstack = partials.reshape(TSC, N, 2)
    out = _nodes(
        x, aggstack, start_features,
        pn_w0[:D_IN], pn_w0[D_IN:], pn_b0.reshape(1, H),
        pn_w1, pn_b1.reshape(1, H), pn_w2, pn_b2.reshape(1, H),
        pn_ln_g.reshape(1, H), pn_ln_b.reshape(1, H),
        nd_w0, nd_b0.reshape(1, H_DEC), nd_w1, nd_b1.reshape(1, H_DEC),
        nd_w2, nd_b2.reshape(1, D_OUT))
    return out


# speedup vs baseline: 2.6057x; 2.6057x over previous
import functools

import jax
import jax.numpy as jnp
from jax import lax
from jax.experimental import pallas as pl
from jax.experimental.pallas import tpu as pltpu
from jax.experimental.pallas import tpu_sc as plsc

N = 10000
E = 160000
D_IN = 256
D_OUT = 78
H = 256
H_DEC = 128

NC = 2
NS = 16
NW = NC * NS
HP = H // 2


BN = 2000


def _pack_bf16(t):
    u = lax.bitcast_convert_type(t, jnp.int32) + 0x8000
    lo = lax.shift_right_logical(u[:, :HP], 16)
    hi = jnp.bitwise_and(u[:, HP:], jnp.int32(-65536))
    return jnp.bitwise_or(lo, hi)


def _unpack_bf16(p):
    left = lax.bitcast_convert_type(lax.shift_left(p, 16), jnp.float32)
    right = lax.bitcast_convert_type(
        jnp.bitwise_and(p, jnp.int32(-65536)), jnp.float32)
    return jnp.concatenate([left, right], axis=1)


def _tables_body(x_ref, ws_ref, wd_ref, b_ref, ts_ref, td_ref):
    x = x_ref[...]
    ts_ref[...] = _pack_bf16(
        jnp.dot(x, ws_ref[...], preferred_element_type=jnp.float32))
    td_ref[...] = _pack_bf16(
        jnp.dot(x, wd_ref[...], preferred_element_type=jnp.float32)
        + b_ref[...])


_tables = pl.pallas_call(
    _tables_body,
    grid=(N // BN,),
    in_specs=[
        pl.BlockSpec((BN, D_IN), lambda i: (i, 0)),
        pl.BlockSpec((D_IN, H), lambda i: (0, 0)),
        pl.BlockSpec((D_IN, H), lambda i: (0, 0)),
        pl.BlockSpec((1, H), lambda i: (0, 0)),
    ],
    out_specs=[
        pl.BlockSpec((BN, HP), lambda i: (i, 0)),
        pl.BlockSpec((BN, HP), lambda i: (i, 0)),
    ],
    out_shape=[
        jax.ShapeDtypeStruct((N, HP), jnp.int32),
        jax.ShapeDtypeStruct((N, HP), jnp.int32),
    ],
)


EW = E // NW
KG = 40
NIT = EW // KG


@functools.partial(
    pl.kernel,
    out_type=[
        jax.ShapeDtypeStruct((E, HP), jnp.int32),
        jax.ShapeDtypeStruct((E, HP), jnp.int32),
    ],
    mesh=plsc.VectorSubcoreMesh(core_axis_name="c", subcore_axis_name="s"),
    scratch_types=[
        pltpu.VMEM((KG,), jnp.int32),
        pltpu.VMEM((KG,), jnp.int32),
        pltpu.VMEM((KG, HP), jnp.int32),
        pltpu.VMEM((KG, HP), jnp.int32),
        pltpu.SemaphoreType.DMA,
        pltpu.SemaphoreType.DMA,
    ],
    compiler_params=pltpu.CompilerParams(needs_layout_passes=False),
)
def _gather_sc(src_hbm, dst_hbm, ts_hbm, td_hbm, gs_hbm, gd_hbm,
               idxs_v, idxd_v, bs_v, bd_v, sem1, sem2):
    wid = lax.axis_index("s") * NC + lax.axis_index("c")
    base = wid * EW

    def body(i, carry):
        b = base + i * KG
        pltpu.sync_copy(src_hbm.at[pl.ds(b, KG)], idxs_v)
        pltpu.sync_copy(dst_hbm.at[pl.ds(b, KG)], idxd_v)
        cp1 = pltpu.async_copy(ts_hbm.at[idxs_v], bs_v, sem1)
        cp2 = pltpu.async_copy(td_hbm.at[idxd_v], bd_v, sem2)
        cp1.wait()
        cp2.wait()
        pltpu.sync_copy(bs_v, gs_hbm.at[pl.ds(b, KG)])
        pltpu.sync_copy(bd_v, gd_hbm.at[pl.ds(b, KG)])
        return carry

    lax.fori_loop(0, NIT, body, 0)


BE = 2000


def _edges_body(attr_ref, gs_ref, gd_ref, dst_ref,
                ee_w0_ref, ee_b0_ref, ee_w1_ref, ee_b1_ref, ee_w2_ref,
                ee_b2_ref, ee_g_ref, ee_bb_ref,
                we_ref, pe_w1_ref, pe_b1_ref, pe_w2_ref,
                pe_b2_ref, pe_g_ref, pe_bb_ref,
                e2_ref, idx2_ref):
    def silu(v):
        return v * jax.nn.sigmoid(v)

    a = attr_ref[...]
    w0 = ee_w0_ref[...]
    h = silu(a[:, 0:1] * w0[0:1, :] + a[:, 1:2] * w0[1:2, :] + ee_b0_ref[...])
    h = silu(jnp.dot(h, ee_w1_ref[...], preferred_element_type=jnp.float32)
             + ee_b1_ref[...])
    epre = (jnp.dot(h, ee_w2_ref[...], preferred_element_type=jnp.float32)
            + ee_b2_ref[...])
    m = (epre[:, 0:1] + epre[:, 1:2]) * 0.5
    d0 = epre[:, 0:1] - m
    r = lax.rsqrt(d0 * d0 + 1e-5)
    g = ee_g_ref[...]
    bb = ee_bb_ref[...]
    e0 = d0 * r * g[:, 0:1] + bb[:, 0:1]
    e1 = -d0 * r * g[:, 1:2] + bb[:, 1:2]

    we = we_ref[...]
    h2 = silu(e0 * we[0:1, :] + e1 * we[1:2, :]
              + _unpack_bf16(gs_ref[...]) + _unpack_bf16(gd_ref[...]))
    h2 = silu(jnp.dot(h2, pe_w1_ref[...], preferred_element_type=jnp.float32)
              + pe_b1_ref[...])
    q = (jnp.dot(h2, pe_w2_ref[...], preferred_element_type=jnp.float32)
         + pe_b2_ref[...])
    m2 = (q[:, 0:1] + q[:, 1:2]) * 0.5
    dq = q[:, 0:1] - m2
    r2 = lax.rsqrt(dq * dq + 1e-5)
    g2 = pe_g_ref[...]
    bb2 = pe_bb_ref[...]
    e2_0 = dq * r2 * g2[:, 0:1] + bb2[:, 0:1] + e0
    e2_1 = -dq * r2 * g2[:, 1:2] + bb2[:, 1:2] + e1
    e2_ref[...] = jnp.concatenate([e2_0, e2_1], axis=1)

    d = dst_ref[...]
    idx2_ref[...] = 2 * d + lax.broadcasted_iota(jnp.int32, (BE, 2), 1)


def _w_spec(shape):
    return pl.BlockSpec(shape, lambda i: tuple(0 for _ in shape))


_edges = pl.pallas_call(
    _edges_body,
    grid=(E // BE,),
    in_specs=[
        pl.BlockSpec((BE, 2), lambda i: (i, 0)),
        pl.BlockSpec((BE, HP), lambda i: (i, 0)),
        pl.BlockSpec((BE, HP), lambda i: (i, 0)),
        pl.BlockSpec((BE, 1), lambda i: (i, 0)),
        _w_spec((2, H)), _w_spec((1, H)), _w_spec((H, H)), _w_spec((1, H)),
        _w_spec((H, 2)), _w_spec((1, 2)), _w_spec((1, 2)), _w_spec((1, 2)),
        _w_spec((2, H)), _w_spec((H, H)), _w_spec((1, H)),
        _w_spec((H, 2)), _w_spec((1, 2)), _w_spec((1, 2)), _w_spec((1, 2)),
    ],
    out_specs=[
        pl.BlockSpec((BE, 2), lambda i: (i, 0)),
        pl.BlockSpec((BE, 2), lambda i: (i, 0)),
    ],
    out_shape=[
        jax.ShapeDtypeStruct((E, 2), jnp.float32),
        jax.ShapeDtypeStruct((E, 2), jnp.int32),
    ],
)


TSC = 8
CH = 2 * E // TSC
SUB = 2000
NSUB = CH // SUB


@functools.partial(
    pl.kernel,
    out_type=jax.ShapeDtypeStruct((TSC, 2 * N), jnp.float32),
    mesh=plsc.VectorSubcoreMesh(core_axis_name="c", subcore_axis_name="s"),
    scratch_types=[
        pltpu.VMEM((SUB,), jnp.int32),
        pltpu.VMEM((SUB,), jnp.float32),
        pltpu.VMEM((2 * N,), jnp.float32),
    ],
    compiler_params=pltpu.CompilerParams(needs_layout_passes=False),
)
def _scatter_sc(idx_hbm, val_hbm, out_hbm, idx_v, val_v, acc_v):
    wid = lax.axis_index("s") * NC + lax.axis_index("c")

    @pl.when(wid < TSC)
    def _():
        def zero(i, carry):
            acc_v[pl.ds(i * 16, 16)] = jnp.zeros((16,), jnp.float32)
            return carry

        lax.fori_loop(0, (2 * N) // 16, zero, 0)

        def sub(s, carry):
            b = wid * CH + s * SUB
            pltpu.sync_copy(idx_hbm.at[pl.ds(b, SUB)], idx_v)
            pltpu.sync_copy(val_hbm.at[pl.ds(b, SUB)], val_v)

            def inner(j, c2):
                iv = idx_v[pl.ds(j * 16, 16)]
                vv = val_v[pl.ds(j * 16, 16)]
                plsc.addupdate_scatter(acc_v, [iv], vv)
                return c2

            lax.fori_loop(0, SUB // 16, inner, 0)
            return carry

        lax.fori_loop(0, NSUB, sub, 0)
        pltpu.sync_copy(acc_v, out_hbm.at[wid])


def _nodes_body(x_ref, agg_ref, st_ref,
                wx_ref, wa_ref, pn_b0_ref, pn_w1_ref, pn_b1_ref,
                pn_w2_ref, pn_b2_ref, pn_g_ref, pn_bb_ref,
                nd_w0_ref, nd_b0_ref, nd_w1_ref, nd_b1_ref,
                nd_w2_ref, nd_b2_ref, out_ref):
    def silu(v):
        return v * jax.nn.sigmoid(v)

    agg = agg_ref[0]
    for k in range(1, TSC):
        agg = agg + agg_ref[k]
    wa = wa_ref[...]
    x = x_ref[...]
    h = silu(jnp.dot(x, wx_ref[...], preferred_element_type=jnp.float32)
             + agg[:, 0:1] * wa[0:1, :] + agg[:, 1:2] * wa[1:2, :]
             + pn_b0_ref[...])
    h = silu(jnp.dot(h, pn_w1_ref[...], preferred_element_type=jnp.float32)
             + pn_b1_ref[...])
    xp = (jnp.dot(h, pn_w2_ref[...], preferred_element_type=jnp.float32)
          + pn_b2_ref[...])
    mu = jnp.mean(xp, axis=-1, keepdims=True)
    ctr = xp - mu
    va = jnp.mean(ctr * ctr, axis=-1, keepdims=True)
    x2 = ctr * lax.rsqrt(va + 1e-5) * pn_g_ref[...] + pn_bb_ref[...] + x
    dd = silu(jnp.dot(x2, nd_w0_ref[...], preferred_element_type=jnp.float32)
              + nd_b0_ref[...])
    dd = silu(jnp.dot(dd, nd_w1_ref[...], preferred_element_type=jnp.float32)
              + nd_b1_ref[...])
    out_ref[...] = (jnp.dot(dd, nd_w2_ref[...],
                            preferred_element_type=jnp.float32)
                    + nd_b2_ref[...] + st_ref[...])


_nodes = pl.pallas_call(
    _nodes_body,
    grid=(N // BN,),
    in_specs=[
        pl.BlockSpec((BN, D_IN), lambda i: (i, 0)),
        pl.BlockSpec((TSC, BN, 2), lambda i: (0, i, 0)),
        pl.BlockSpec((BN, D_OUT), lambda i: (i, 0)),
        _w_spec((D_IN, H)), _w_spec((2, H)), _w_spec((1, H)),
        _w_spec((H, H)), _w_spec((1, H)),
        _w_spec((H, D_IN)), _w_spec((1, D_IN)), _w_spec((1, D_IN)),
        _w_spec((1, D_IN)),
        _w_spec((D_IN, H_DEC)), _w_spec((1, H_DEC)),
        _w_spec((H_DEC, H_DEC)), _w_spec((1, H_DEC)),
        _w_spec((H_DEC, D_OUT)), _w_spec((1, D_OUT)),
    ],
    out_specs=pl.BlockSpec((BN, D_OUT), lambda i: (i, 0)),
    out_shape=jax.ShapeDtypeStruct((N, D_OUT), jnp.float32),
)


def kernel(processor_features, start_features, edge_attr, edge_index,
           ee_w0, ee_b0, ee_w1, ee_b1, ee_w2, ee_b2, ee_ln_g, ee_ln_b,
           pe_w0, pe_b0, pe_w1, pe_b1, pe_w2, pe_b2, pe_ln_g, pe_ln_b,
           pn_w0, pn_b0, pn_w1, pn_b1, pn_w2, pn_b2, pn_ln_g, pn_ln_b,
           nd_w0, nd_b0, nd_w1, nd_b1, nd_w2, nd_b2):
    x = processor_features
    src = edge_index[0]
    dst = edge_index[1]
    we = pe_w0[0:2]
    ws = pe_w0[2:2 + D_IN]
    wd = pe_w0[2 + D_IN:2 + 2 * D_IN]

    ts_p, td_p = _tables(x, ws, wd, pe_b0.reshape(1, H))
    gs_p, gd_p = _gather_sc(src, dst, ts_p, td_p)
    e2, idx2 = _edges(
        edge_attr, gs_p, gd_p, dst.reshape(E, 1),
        ee_w0, ee_b0.reshape(1, H), ee_w1, ee_b1.reshape(1, H),
        ee_w2, ee_b2.reshape(1, 2), ee_ln_g.reshape(1, 2),
        ee_ln_b.reshape(1, 2),
        we, pe_w1, pe_b1.reshape(1, H), pe_w2, pe_b2.reshape(1, 2),
        pe_ln_g.reshape(1, 2), pe_ln_b.reshape(1, 2))
    partials = _scatter_sc(idx2.reshape(2 * E), e2.reshape(2 * E))
    aggstack = partials.reshape(TSC, N, 2)
    out = _nodes(
        x, aggstack, start_features,
        pn_w0[:D_IN], pn_w0[D_IN:], pn_b0.reshape(1, H),
        pn_w1, pn_b1.reshape(1, H), pn_w2, pn_b2.reshape(1, H),
        pn_ln_g.reshape(1, H), pn_ln_b.reshape(1, H),
        nd_w0, nd_b0.reshape(1, H_DEC), nd_w1, nd_b1.reshape(1, H_DEC),
        nd_w2, nd_b2.reshape(1, D_OUT))
    return out


# speedup vs baseline: 3.0021x; 1.1521x over previous
import functools

import jax
import jax.numpy as jnp
from jax import lax
from jax.experimental import pallas as pl
from jax.experimental.pallas import tpu as pltpu
from jax.experimental.pallas import tpu_sc as plsc

N = 10000
E = 160000
D_IN = 256
D_OUT = 78
H = 256
H_DEC = 128

NC = 2
NS = 16
NW = NC * NS
HP = H // 2

NSL = 5
ES = E // NSL


def _pack_bf16(t):
    u = lax.bitcast_convert_type(t, jnp.int32) + 0x8000
    lo = lax.shift_right_logical(u[:, :HP], 16)
    hi = jnp.bitwise_and(u[:, HP:], jnp.int32(-65536))
    return jnp.bitwise_or(lo, hi)


def _unpack_bf16(p):
    left = lax.bitcast_convert_type(lax.shift_left(p, 16), jnp.float32)
    right = lax.bitcast_convert_type(
        jnp.bitwise_and(p, jnp.int32(-65536)), jnp.float32)
    return jnp.concatenate([left, right], axis=1)


BN = 2000


def _tables_body(x_ref, ws_ref, wd_ref, b_ref, ts_ref, td_ref):
    x = x_ref[...]
    ts_ref[...] = _pack_bf16(
        jnp.dot(x, ws_ref[...], preferred_element_type=jnp.float32))
    td_ref[...] = _pack_bf16(
        jnp.dot(x, wd_ref[...], preferred_element_type=jnp.float32)
        + b_ref[...])


_tables = pl.pallas_call(
    _tables_body,
    grid=(N // BN,),
    in_specs=[
        pl.BlockSpec((BN, D_IN), lambda i: (i, 0)),
        pl.BlockSpec((D_IN, H), lambda i: (0, 0)),
        pl.BlockSpec((D_IN, H), lambda i: (0, 0)),
        pl.BlockSpec((1, H), lambda i: (0, 0)),
    ],
    out_specs=[
        pl.BlockSpec((BN, HP), lambda i: (i, 0)),
        pl.BlockSpec((BN, HP), lambda i: (i, 0)),
    ],
    out_shape=[
        jax.ShapeDtypeStruct((N, HP), jnp.int32),
        jax.ShapeDtypeStruct((N, HP), jnp.int32),
    ],
)


EWS = ES // NW
KG = 40
NIT = EWS // KG


@functools.partial(
    pl.kernel,
    out_type=[
        jax.ShapeDtypeStruct((ES, HP), jnp.int32),
        jax.ShapeDtypeStruct((ES, HP), jnp.int32),
    ],
    mesh=plsc.VectorSubcoreMesh(core_axis_name="c", subcore_axis_name="s"),
    scratch_types=[
        pltpu.VMEM((KG,), jnp.int32),
        pltpu.VMEM((KG,), jnp.int32),
        pltpu.VMEM((KG, HP), jnp.int32),
        pltpu.VMEM((KG, HP), jnp.int32),
        pltpu.SemaphoreType.DMA,
        pltpu.SemaphoreType.DMA,
    ],
    compiler_params=pltpu.CompilerParams(needs_layout_passes=False),
)
def _gather_sc(src_hbm, dst_hbm, ts_hbm, td_hbm, gs_hbm, gd_hbm,
               idxs_v, idxd_v, bs_v, bd_v, sem1, sem2):
    wid = lax.axis_index("s") * NC + lax.axis_index("c")
    base = wid * EWS

    def body(i, carry):
        b = base + i * KG
        pltpu.sync_copy(src_hbm.at[pl.ds(b, KG)], idxs_v)
        pltpu.sync_copy(dst_hbm.at[pl.ds(b, KG)], idxd_v)
        cp1 = pltpu.async_copy(ts_hbm.at[idxs_v], bs_v, sem1)
        cp2 = pltpu.async_copy(td_hbm.at[idxd_v], bd_v, sem2)
        cp1.wait()
        cp2.wait()
        pltpu.sync_copy(bs_v, gs_hbm.at[pl.ds(b, KG)])
        pltpu.sync_copy(bd_v, gd_hbm.at[pl.ds(b, KG)])
        return carry

    lax.fori_loop(0, NIT, body, 0)


BE = 2000


def _edges_body(attr_ref, gs_ref, gd_ref, dst_ref,
                ee_w0_ref, ee_b0_ref, ee_w1_ref, ee_b1_ref, ee_w2_ref,
                ee_b2_ref, ee_g_ref, ee_bb_ref,
                we_ref, pe_w1_ref, pe_b1_ref, pe_w2_ref,
                pe_b2_ref, pe_g_ref, pe_bb_ref,
                e2_ref, idx2_ref):
    def silu(v):
        return v * jax.nn.sigmoid(v)

    a = attr_ref[...]
    w0 = ee_w0_ref[...]
    h = silu(a[:, 0:1] * w0[0:1, :] + a[:, 1:2] * w0[1:2, :] + ee_b0_ref[...])
    h = silu(jnp.dot(h, ee_w1_ref[...], preferred_element_type=jnp.float32)
             + ee_b1_ref[...])
    epre = (jnp.dot(h, ee_w2_ref[...], preferred_element_type=jnp.float32)
            + ee_b2_ref[...])
    m = (epre[:, 0:1] + epre[:, 1:2]) * 0.5
    d0 = epre[:, 0:1] - m
    r = lax.rsqrt(d0 * d0 + 1e-5)
    g = ee_g_ref[...]
    bb = ee_bb_ref[...]
    e0 = d0 * r * g[:, 0:1] + bb[:, 0:1]
    e1 = -d0 * r * g[:, 1:2] + bb[:, 1:2]

    we = we_ref[...]
    h2 = silu(e0 * we[0:1, :] + e1 * we[1:2, :]
              + _unpack_bf16(gs_ref[...]) + _unpack_bf16(gd_ref[...]))
    h2 = silu(jnp.dot(h2, pe_w1_ref[...], preferred_element_type=jnp.float32)
              + pe_b1_ref[...])
    q = (jnp.dot(h2, pe_w2_ref[...], preferred_element_type=jnp.float32)
         + pe_b2_ref[...])
    m2 = (q[:, 0:1] + q[:, 1:2]) * 0.5
    dq = q[:, 0:1] - m2
    r2 = lax.rsqrt(dq * dq + 1e-5)
    g2 = pe_g_ref[...]
    bb2 = pe_bb_ref[...]
    e2_0 = dq * r2 * g2[:, 0:1] + bb2[:, 0:1] + e0
    e2_1 = -dq * r2 * g2[:, 1:2] + bb2[:, 1:2] + e1
    e2_ref[...] = jnp.concatenate([e2_0, e2_1], axis=1)

    d = dst_ref[...]
    idx2_ref[...] = 2 * d + lax.broadcasted_iota(jnp.int32, (BE, 2), 1)


def _w_spec(shape):
    return pl.BlockSpec(shape, lambda i: tuple(0 for _ in shape))


_edges = pl.pallas_call(
    _edges_body,
    grid=(ES // BE,),
    in_specs=[
        pl.BlockSpec((BE, 2), lambda i: (i, 0)),
        pl.BlockSpec((BE, HP), lambda i: (i, 0)),
        pl.BlockSpec((BE, HP), lambda i: (i, 0)),
        pl.BlockSpec((BE, 1), lambda i: (i, 0)),
        _w_spec((2, H)), _w_spec((1, H)), _w_spec((H, H)), _w_spec((1, H)),
        _w_spec((H, 2)), _w_spec((1, 2)), _w_spec((1, 2)), _w_spec((1, 2)),
        _w_spec((2, H)), _w_spec((H, H)), _w_spec((1, H)),
        _w_spec((H, 2)), _w_spec((1, 2)), _w_spec((1, 2)), _w_spec((1, 2)),
    ],
    out_specs=[
        pl.BlockSpec((BE, 2), lambda i: (i, 0)),
        pl.BlockSpec((BE, 2), lambda i: (i, 0)),
    ],
    out_shape=[
        jax.ShapeDtypeStruct((ES, 2), jnp.float32),
        jax.ShapeDtypeStruct((ES, 2), jnp.int32),
    ],
)


TSC = 4
CH = 2 * ES // TSC
SUB = 2000
NSUB = CH // SUB
NPART = NSL * TSC


@functools.partial(
    pl.kernel,
    out_type=jax.ShapeDtypeStruct((TSC, 2 * N), jnp.float32),
    mesh=plsc.VectorSubcoreMesh(core_axis_name="c", subcore_axis_name="s"),
    scratch_types=[
        pltpu.VMEM((SUB,), jnp.int32),
        pltpu.VMEM((SUB,), jnp.float32),
        pltpu.VMEM((2 * N,), jnp.float32),
    ],
    compiler_params=pltpu.CompilerParams(needs_layout_passes=False),
)
def _scatter_sc(idx_hbm, val_hbm, out_hbm, idx_v, val_v, acc_v):
    wid = lax.axis_index("s") * NC + lax.axis_index("c")

    @pl.when(wid < TSC)
    def _():
        def zero(i, carry):
            acc_v[pl.ds(i * 16, 16)] = jnp.zeros((16,), jnp.float32)
            return carry

        lax.fori_loop(0, (2 * N) // 16, zero, 0)

        def sub(s, carry):
            b = wid * CH + s * SUB
            pltpu.sync_copy(idx_hbm.at[pl.ds(b, SUB)], idx_v)
            pltpu.sync_copy(val_hbm.at[pl.ds(b, SUB)], val_v)

            def inner(j, c2):
                iv = idx_v[pl.ds(j * 16, 16)]
                vv = val_v[pl.ds(j * 16, 16)]
                plsc.addupdate_scatter(acc_v, [iv], vv)
                return c2

            lax.fori_loop(0, SUB // 16, inner, 0)
            return carry

        lax.fori_loop(0, NSUB, sub, 0)
        pltpu.sync_copy(acc_v, out_hbm.at[wid])


def _nodes_body(x_ref, agg_ref, st_ref,
                wx_ref, wa_ref, pn_b0_ref, pn_w1_ref, pn_b1_ref,
                pn_w2_ref, pn_b2_ref, pn_g_ref, pn_bb_ref,
                nd_w0_ref, nd_b0_ref, nd_w1_ref, nd_b1_ref,
                nd_w2_ref, nd_b2_ref, out_ref):
    def silu(v):
        return v * jax.nn.sigmoid(v)

    agg = agg_ref[0]
    for k in range(1, NPART):
        agg = agg + agg_ref[k]
    wa = wa_ref[...]
    x = x_ref[...]
    h = silu(jnp.dot(x, wx_ref[...], preferred_element_type=jnp.float32)
             + agg[:, 0:1] * wa[0:1, :] + agg[:, 1:2] * wa[1:2, :]
             + pn_b0_ref[...])
    h = silu(jnp.dot(h, pn_w1_ref[...], preferred_element_type=jnp.float32)
             + pn_b1_ref[...])
    xp = (jnp.dot(h, pn_w2_ref[...], preferred_element_type=jnp.float32)
          + pn_b2_ref[...])
    mu = jnp.mean(xp, axis=-1, keepdims=True)
    ctr = xp - mu
    va = jnp.mean(ctr * ctr, axis=-1, keepdims=True)
    x2 = ctr * lax.rsqrt(va + 1e-5) * pn_g_ref[...] + pn_bb_ref[...] + x
    dd = silu(jnp.dot(x2, nd_w0_ref[...], preferred_element_type=jnp.float32)
              + nd_b0_ref[...])
    dd = silu(jnp.dot(dd, nd_w1_ref[...], preferred_element_type=jnp.float32)
              + nd_b1_ref[...])
    out_ref[...] = (jnp.dot(dd, nd_w2_ref[...],
                            preferred_element_type=jnp.float32)
                    + nd_b2_ref[...] + st_ref[...])


_nodes = pl.pallas_call(
    _nodes_body,
    grid=(N // BN,),
    in_specs=[
        pl.BlockSpec((BN, D_IN), lambda i: (i, 0)),
        pl.BlockSpec((NPART, BN, 2), lambda i: (0, i, 0)),
        pl.BlockSpec((BN, D_OUT), lambda i: (i, 0)),
        _w_spec((D_IN, H)), _w_spec((2, H)), _w_spec((1, H)),
        _w_spec((H, H)), _w_spec((1, H)),
        _w_spec((H, D_IN)), _w_spec((1, D_IN)), _w_spec((1, D_IN)),
        _w_spec((1, D_IN)),
        _w_spec((D_IN, H_DEC)), _w_spec((1, H_DEC)),
        _w_spec((H_DEC, H_DEC)), _w_spec((1, H_DEC)),
        _w_spec((H_DEC, D_OUT)), _w_spec((1, D_OUT)),
    ],
    out_specs=pl.BlockSpec((BN, D_OUT), lambda i: (i, 0)),
    out_shape=jax.ShapeDtypeStruct((N, D_OUT), jnp.float32),
)


def kernel(processor_features, start_features, edge_attr, edge_index,
           ee_w0, ee_b0, ee_w1, ee_b1, ee_w2, ee_b2, ee_ln_g, ee_ln_b,
           pe_w0, pe_b0, pe_w1, pe_b1, pe_w2, pe_b2, pe_ln_g, pe_ln_b,
           pn_w0, pn_b0, pn_w1, pn_b1, pn_w2, pn_b2, pn_ln_g, pn_ln_b,
           nd_w0, nd_b0, nd_w1, nd_b1, nd_w2, nd_b2):
    x = processor_features
    src = edge_index[0]
    dst = edge_index[1]
    we = pe_w0[0:2]
    ws = pe_w0[2:2 + D_IN]
    wd = pe_w0[2 + D_IN:2 + 2 * D_IN]

    ts_p, td_p = _tables(x, ws, wd, pe_b0.reshape(1, H))

    ew_args = (
        ee_w0, ee_b0.reshape(1, H), ee_w1, ee_b1.reshape(1, H),
        ee_w2, ee_b2.reshape(1, 2), ee_ln_g.reshape(1, 2),
        ee_ln_b.reshape(1, 2),
        we, pe_w1, pe_b1.reshape(1, H), pe_w2, pe_b2.reshape(1, 2),
        pe_ln_g.reshape(1, 2), pe_ln_b.reshape(1, 2))

    parts = []
    for k in range(NSL):
        sl = slice(k * ES, (k + 1) * ES)
        src_k = src[sl]
        dst_k = dst[sl]
        gs_k, gd_k = _gather_sc(src_k, dst_k, ts_p, td_p)
        e2_k, idx2_k = _edges(edge_attr[sl], gs_k, gd_k,
                              dst_k.reshape(ES, 1), *ew_args)
        parts.append(_scatter_sc(idx2_k.reshape(2 * ES),
                                 e2_k.reshape(2 * ES)))

    aggstack = jnp.concatenate(parts, axis=0).reshape(NPART, N, 2)
    out = _nodes(
        x, aggstack, start_features,
        pn_w0[:D_IN], pn_w0[D_IN:], pn_b0.reshape(1, H),
        pn_w1, pn_b1.reshape(1, H), pn_w2, pn_b2.reshape(1, H),
        pn_ln_g.reshape(1, H), pn_ln_b.reshape(1, H),
        nd_w0, nd_b0.reshape(1, H_DEC), nd_w1, nd_b1.reshape(1, H_DEC),
        nd_w2, nd_b2.reshape(1, D_OUT))
    return out


# speedup vs baseline: 3.0121x; 1.0033x over previous
import functools

import jax
import jax.numpy as jnp
from jax import lax
from jax.experimental import pallas as pl
from jax.experimental.pallas import tpu as pltpu
from jax.experimental.pallas import tpu_sc as plsc

N = 10000
E = 160000
D_IN = 256
D_OUT = 78
H = 256
H_DEC = 128

NC = 2
NS = 16
NW = NC * NS
HP = H // 2

NSL = 5
ES = E // NSL


def _pack_bf16(t):
    u = lax.bitcast_convert_type(t, jnp.int32) + 0x8000
    lo = lax.shift_right_logical(u[:, :HP], 16)
    hi = jnp.bitwise_and(u[:, HP:], jnp.int32(-65536))
    return jnp.bitwise_or(lo, hi)


def _unpack_bf16(p):
    left = lax.bitcast_convert_type(lax.shift_left(p, 16), jnp.float32)
    right = lax.bitcast_convert_type(
        jnp.bitwise_and(p, jnp.int32(-65536)), jnp.float32)
    return jnp.concatenate([left, right], axis=1)


BN = 2000


def _tables_body(x_ref, ws_ref, wd_ref, b_ref, ts_ref, td_ref):
    x = x_ref[...]
    ts_ref[...] = _pack_bf16(
        jnp.dot(x, ws_ref[...], preferred_element_type=jnp.float32))
    td_ref[...] = _pack_bf16(
        jnp.dot(x, wd_ref[...], preferred_element_type=jnp.float32)
        + b_ref[...])


_tables = pl.pallas_call(
    _tables_body,
    grid=(N // BN,),
    in_specs=[
        pl.BlockSpec((BN, D_IN), lambda i: (i, 0)),
        pl.BlockSpec((D_IN, H), lambda i: (0, 0)),
        pl.BlockSpec((D_IN, H), lambda i: (0, 0)),
        pl.BlockSpec((1, H), lambda i: (0, 0)),
    ],
    out_specs=[
        pl.BlockSpec((BN, HP), lambda i: (i, 0)),
        pl.BlockSpec((BN, HP), lambda i: (i, 0)),
    ],
    out_shape=[
        jax.ShapeDtypeStruct((N, HP), jnp.int32),
        jax.ShapeDtypeStruct((N, HP), jnp.int32),
    ],
)


EWS = ES // NW
KG = 128
KT = EWS - (EWS // KG) * KG
NFULL = EWS // KG


@functools.partial(
    pl.kernel,
    out_type=[
        jax.ShapeDtypeStruct((ES, HP), jnp.int32),
        jax.ShapeDtypeStruct((ES, HP), jnp.int32),
    ],
    mesh=plsc.VectorSubcoreMesh(core_axis_name="c", subcore_axis_name="s"),
    scratch_types=[
        pltpu.VMEM((EWS,), jnp.int32),
        pltpu.VMEM((EWS,), jnp.int32),
        pltpu.VMEM((KG, HP), jnp.int32),
        pltpu.VMEM((KG, HP), jnp.int32),
        pltpu.SemaphoreType.DMA,
        pltpu.SemaphoreType.DMA,
    ],
    compiler_params=pltpu.CompilerParams(needs_layout_passes=False),
)
def _gather_sc(src_hbm, dst_hbm, ts_hbm, td_hbm, gs_hbm, gd_hbm,
               idxs_v, idxd_v, bs_v, bd_v, sem1, sem2):
    wid = lax.axis_index("s") * NC + lax.axis_index("c")
    base = wid * EWS
    pltpu.sync_copy(src_hbm.at[pl.ds(base, EWS)], idxs_v)
    pltpu.sync_copy(dst_hbm.at[pl.ds(base, EWS)], idxd_v)

    def chunk(off, k):
        cp1 = pltpu.async_copy(
            ts_hbm.at[idxs_v.at[pl.ds(off, k)]], bs_v.at[pl.ds(0, k)], sem1)
        cp2 = pltpu.async_copy(
            td_hbm.at[idxd_v.at[pl.ds(off, k)]], bd_v.at[pl.ds(0, k)], sem2)
        cp1.wait()
        cp2.wait()
        sl = pl.ds(base + off, k)
        pltpu.sync_copy(bs_v.at[pl.ds(0, k)], gs_hbm.at[sl])
        pltpu.sync_copy(bd_v.at[pl.ds(0, k)], gd_hbm.at[sl])

    def body(i, carry):
        chunk(i * KG, KG)
        return carry

    lax.fori_loop(0, NFULL, body, 0)
    chunk(NFULL * KG, KT)


BE = 2000


def _edges_body(attr_ref, gs_ref, gd_ref, dst_ref,
                ee_w0_ref, ee_b0_ref, ee_w1_ref, ee_b1_ref, ee_w2_ref,
                ee_b2_ref, ee_g_ref, ee_bb_ref,
                we_ref, pe_w1_ref, pe_b1_ref, pe_w2_ref,
                pe_b2_ref, pe_g_ref, pe_bb_ref,
                e2_ref, idx2_ref):
    def silu(v):
        return v * jax.nn.sigmoid(v)

    a = attr_ref[...]
    w0 = ee_w0_ref[...]
    h = silu(a[:, 0:1] * w0[0:1, :] + a[:, 1:2] * w0[1:2, :] + ee_b0_ref[...])
    h = silu(jnp.dot(h, ee_w1_ref[...], preferred_element_type=jnp.float32)
             + ee_b1_ref[...])
    epre = (jnp.dot(h, ee_w2_ref[...], preferred_element_type=jnp.float32)
            + ee_b2_ref[...])
    m = (epre[:, 0:1] + epre[:, 1:2]) * 0.5
    d0 = epre[:, 0:1] - m
    r = lax.rsqrt(d0 * d0 + 1e-5)
    g = ee_g_ref[...]
    bb = ee_bb_ref[...]
    e0 = d0 * r * g[:, 0:1] + bb[:, 0:1]
    e1 = -d0 * r * g[:, 1:2] + bb[:, 1:2]

    we = we_ref[...]
    h2 = silu(e0 * we[0:1, :] + e1 * we[1:2, :]
              + _unpack_bf16(gs_ref[...]) + _unpack_bf16(gd_ref[...]))
    h2 = silu(jnp.dot(h2, pe_w1_ref[...], preferred_element_type=jnp.float32)
              + pe_b1_ref[...])
    q = (jnp.dot(h2, pe_w2_ref[...], preferred_element_type=jnp.float32)
         + pe_b2_ref[...])
    m2 = (q[:, 0:1] + q[:, 1:2]) * 0.5
    dq = q[:, 0:1] - m2
    r2 = lax.rsqrt(dq * dq + 1e-5)
    g2 = pe_g_ref[...]
    bb2 = pe_bb_ref[...]
    e2_0 = dq * r2 * g2[:, 0:1] + bb2[:, 0:1] + e0
    e2_1 = -dq * r2 * g2[:, 1:2] + bb2[:, 1:2] + e1
    e2_ref[...] = jnp.concatenate([e2_0, e2_1], axis=1)

    d = dst_ref[...]
    idx2_ref[...] = 2 * d + lax.broadcasted_iota(jnp.int32, (BE, 2), 1)


def _w_spec(shape):
    return pl.BlockSpec(shape, lambda i: tuple(0 for _ in shape))


_edges = pl.pallas_call(
    _edges_body,
    grid=(ES // BE,),
    in_specs=[
        pl.BlockSpec((BE, 2), lambda i: (i, 0)),
        pl.BlockSpec((BE, HP), lambda i: (i, 0)),
        pl.BlockSpec((BE, HP), lambda i: (i, 0)),
        pl.BlockSpec((BE, 1), lambda i: (i, 0)),
        _w_spec((2, H)), _w_spec((1, H)), _w_spec((H, H)), _w_spec((1, H)),
        _w_spec((H, 2)), _w_spec((1, 2)), _w_spec((1, 2)), _w_spec((1, 2)),
        _w_spec((2, H)), _w_spec((H, H)), _w_spec((1, H)),
        _w_spec((H, 2)), _w_spec((1, 2)), _w_spec((1, 2)), _w_spec((1, 2)),
    ],
    out_specs=[
        pl.BlockSpec((BE, 2), lambda i: (i, 0)),
        pl.BlockSpec((BE, 2), lambda i: (i, 0)),
    ],
    out_shape=[
        jax.ShapeDtypeStruct((ES, 2), jnp.float32),
        jax.ShapeDtypeStruct((ES, 2), jnp.int32),
    ],
)


TSC = 4
CH = 2 * ES // TSC
SUB = 2000
NSUB = CH // SUB
NPART = NSL * TSC


@functools.partial(
    pl.kernel,
    out_type=jax.ShapeDtypeStruct((TSC, 2 * N), jnp.float32),
    mesh=plsc.VectorSubcoreMesh(core_axis_name="c", subcore_axis_name="s"),
    scratch_types=[
        pltpu.VMEM((SUB,), jnp.int32),
        pltpu.VMEM((SUB,), jnp.float32),
        pltpu.VMEM((2 * N,), jnp.float32),
    ],
    compiler_params=pltpu.CompilerParams(needs_layout_passes=False),
)
def _scatter_sc(idx_hbm, val_hbm, out_hbm, idx_v, val_v, acc_v):
    wid = lax.axis_index("s") * NC + lax.axis_index("c")

    @pl.when(wid < TSC)
    def _():
        def zero(i, carry):
            acc_v[pl.ds(i * 16, 16)] = jnp.zeros((16,), jnp.float32)
            return carry

        lax.fori_loop(0, (2 * N) // 16, zero, 0)

        def sub(s, carry):
            b = wid * CH + s * SUB
            pltpu.sync_copy(idx_hbm.at[pl.ds(b, SUB)], idx_v)
            pltpu.sync_copy(val_hbm.at[pl.ds(b, SUB)], val_v)

            def inner(j, c2):
                iv = idx_v[pl.ds(j * 16, 16)]
                vv = val_v[pl.ds(j * 16, 16)]
                plsc.addupdate_scatter(acc_v, [iv], vv)
                return c2

            lax.fori_loop(0, SUB // 16, inner, 0)
            return carry

        lax.fori_loop(0, NSUB, sub, 0)
        pltpu.sync_copy(acc_v, out_hbm.at[wid])


def _nodes_body(x_ref, agg_ref, st_ref,
                wx_ref, wa_ref, pn_b0_ref, pn_w1_ref, pn_b1_ref,
                pn_w2_ref, pn_b2_ref, pn_g_ref, pn_bb_ref,
                nd_w0_ref, nd_b0_ref, nd_w1_ref, nd_b1_ref,
                nd_w2_ref, nd_b2_ref, out_ref):
    def silu(v):
        return v * jax.nn.sigmoid(v)

    agg = agg_ref[0]
    for k in range(1, NPART):
        agg = agg + agg_ref[k]
    wa = wa_ref[...]
    x = x_ref[...]
    h = silu(jnp.dot(x, wx_ref[...], preferred_element_type=jnp.float32)
             + agg[:, 0:1] * wa[0:1, :] + agg[:, 1:2] * wa[1:2, :]
             + pn_b0_ref[...])
    h = silu(jnp.dot(h, pn_w1_ref[...], preferred_element_type=jnp.float32)
             + pn_b1_ref[...])
    xp = (jnp.dot(h, pn_w2_ref[...], preferred_element_type=jnp.float32)
          + pn_b2_ref[...])
    mu = jnp.mean(xp, axis=-1, keepdims=True)
    ctr = xp - mu
    va = jnp.mean(ctr * ctr, axis=-1, keepdims=True)
    x2 = ctr * lax.rsqrt(va + 1e-5) * pn_g_ref[...] + pn_bb_ref[...] + x
    dd = silu(jnp.dot(x2, nd_w0_ref[...], preferred_element_type=jnp.float32)
              + nd_b0_ref[...])
    dd = silu(jnp.dot(dd, nd_w1_ref[...], preferred_element_type=jnp.float32)
              + nd_b1_ref[...])
    out_ref[...] = (jnp.dot(dd, nd_w2_ref[...],
                            preferred_element_type=jnp.float32)
                    + nd_b2_ref[...] + st_ref[...])


_nodes = pl.pallas_call(
    _nodes_body,
    grid=(N // BN,),
    in_specs=[
        pl.BlockSpec((BN, D_IN), lambda i: (i, 0)),
        pl.BlockSpec((NPART, BN, 2), lambda i: (0, i, 0)),
        pl.BlockSpec((BN, D_OUT), lambda i: (i, 0)),
        _w_spec((D_IN, H)), _w_spec((2, H)), _w_spec((1, H)),
        _w_spec((H, H)), _w_spec((1, H)),
        _w_spec((H, D_IN)), _w_spec((1, D_IN)), _w_spec((1, D_IN)),
        _w_spec((1, D_IN)),
        _w_spec((D_IN, H_DEC)), _w_spec((1, H_DEC)),
        _w_spec((H_DEC, H_DEC)), _w_spec((1, H_DEC)),
        _w_spec((H_DEC, D_OUT)), _w_spec((1, D_OUT)),
    ],
    out_specs=pl.BlockSpec((BN, D_OUT), lambda i: (i, 0)),
    out_shape=jax.ShapeDtypeStruct((N, D_OUT), jnp.float32),
)


def kernel(processor_features, start_features, edge_attr, edge_index,
           ee_w0, ee_b0, ee_w1, ee_b1, ee_w2, ee_b2, ee_ln_g, ee_ln_b,
           pe_w0, pe_b0, pe_w1, pe_b1, pe_w2, pe_b2, pe_ln_g, pe_ln_b,
           pn_w0, pn_b0, pn_w1, pn_b1, pn_w2, pn_b2, pn_ln_g, pn_ln_b,
           nd_w0, nd_b0, nd_w1, nd_b1, nd_w2, nd_b2):
    x = processor_features
    src = edge_index[0]
    dst = edge_index[1]
    we = pe_w0[0:2]
    ws = pe_w0[2:2 + D_IN]
    wd = pe_w0[2 + D_IN:2 + 2 * D_IN]

    ts_p, td_p = _tables(x, ws, wd, pe_b0.reshape(1, H))

    ew_args = (
        ee_w0, ee_b0.reshape(1, H), ee_w1, ee_b1.reshape(1, H),
        ee_w2, ee_b2.reshape(1, 2), ee_ln_g.reshape(1, 2),
        ee_ln_b.reshape(1, 2),
        we, pe_w1, pe_b1.reshape(1, H), pe_w2, pe_b2.reshape(1, 2),
        pe_ln_g.reshape(1, 2), pe_ln_b.reshape(1, 2))

    parts = []
    for k in range(NSL):
        sl = slice(k * ES, (k + 1) * ES)
        src_k = src[sl]
        dst_k = dst[sl]
        gs_k, gd_k = _gather_sc(src_k, dst_k, ts_p, td_p)
        e2_k, idx2_k = _edges(edge_attr[sl], gs_k, gd_k,
                              dst_k.reshape(ES, 1), *ew_args)
        parts.append(_scatter_sc(idx2_k.reshape(2 * ES),
                                 e2_k.reshape(2 * ES)))

    aggstack = jnp.concatenate(parts, axis=0).reshape(NPART, N, 2)
    out = _nodes(
        x, aggstack, start_features,
        pn_w0[:D_IN], pn_w0[D_IN:], pn_b0.reshape(1, H),
        pn_w1, pn_b1.reshape(1, H), pn_w2, pn_b2.reshape(1, H),
        pn_ln_g.reshape(1, H), pn_ln_b.reshape(1, H),
        nd_w0, nd_b0.reshape(1, H_DEC), nd_w1, nd_b1.reshape(1, H_DEC),
        nd_w2, nd_b2.reshape(1, D_OUT))
    return out


# speedup vs baseline: 5.5313x; 1.8364x over previous
import functools

import jax
import jax.numpy as jnp
from jax import lax
from jax.experimental import pallas as pl
from jax.experimental.pallas import tpu as pltpu
from jax.experimental.pallas import tpu_sc as plsc

N = 10000
E = 160000
D_IN = 256
D_OUT = 78
H = 256
H_DEC = 128

NC = 2
NS = 16
NW = NC * NS
HP = H // 2

NSL = 5
ES = E // NSL


def _pack_bf16(t):
    u = lax.bitcast_convert_type(t, jnp.int32) + 0x8000
    lo = lax.shift_right_logical(u[:, :HP], 16)
    hi = jnp.bitwise_and(u[:, HP:], jnp.int32(-65536))
    return jnp.bitwise_or(lo, hi)


def _unpack_bf16(p):
    left = lax.bitcast_convert_type(lax.shift_left(p, 16), jnp.float32)
    right = lax.bitcast_convert_type(
        jnp.bitwise_and(p, jnp.int32(-65536)), jnp.float32)
    return jnp.concatenate([left, right], axis=1)


BN = 2000


def _tables_body(x_ref, ws_ref, wd_ref, b_ref, ts_ref, td_ref):
    x = x_ref[...]
    ts_ref[...] = _pack_bf16(
        jnp.dot(x, ws_ref[...], preferred_element_type=jnp.float32))
    td_ref[...] = _pack_bf16(
        jnp.dot(x, wd_ref[...], preferred_element_type=jnp.float32)
        + b_ref[...])


_tables = pl.pallas_call(
    _tables_body,
    grid=(N // BN,),
    in_specs=[
        pl.BlockSpec((BN, D_IN), lambda i: (i, 0)),
        pl.BlockSpec((D_IN, H), lambda i: (0, 0)),
        pl.BlockSpec((D_IN, H), lambda i: (0, 0)),
        pl.BlockSpec((1, H), lambda i: (0, 0)),
    ],
    out_specs=[
        pl.BlockSpec((BN, HP), lambda i: (i, 0)),
        pl.BlockSpec((BN, HP), lambda i: (i, 0)),
    ],
    out_shape=[
        jax.ShapeDtypeStruct((N, HP), jnp.int32),
        jax.ShapeDtypeStruct((N, HP), jnp.int32),
    ],
)


EWS = ES // NW
KG = 128
KT = EWS - (EWS // KG) * KG
NFULL = EWS // KG


def _make_gather(slice_k):
    sbase = slice_k * ES

    @functools.partial(
        pl.kernel,
        out_type=[
            jax.ShapeDtypeStruct((ES, HP), jnp.int32),
            jax.ShapeDtypeStruct((ES, HP), jnp.int32),
        ],
        mesh=plsc.VectorSubcoreMesh(core_axis_name="c", subcore_axis_name="s"),
        scratch_types=[
            pltpu.VMEM((EWS,), jnp.int32),
            pltpu.VMEM((EWS,), jnp.int32),
            pltpu.VMEM((KG, HP), jnp.int32),
            pltpu.VMEM((KG, HP), jnp.int32),
            pltpu.SemaphoreType.DMA,
            pltpu.SemaphoreType.DMA,
        ],
        compiler_params=pltpu.CompilerParams(needs_layout_passes=False),
    )
    def gather(ei_hbm, ts_hbm, td_hbm, gs_hbm, gd_hbm,
               idxs_v, idxd_v, bs_v, bd_v, sem1, sem2):
        wid = lax.axis_index("s") * NC + lax.axis_index("c")
        base = wid * EWS
        pltpu.sync_copy(ei_hbm.at[pl.ds(sbase + base, EWS)], idxs_v)
        pltpu.sync_copy(ei_hbm.at[pl.ds(E + sbase + base, EWS)], idxd_v)

        def chunk(off, k):
            cp1 = pltpu.async_copy(
                ts_hbm.at[idxs_v.at[pl.ds(off, k)]],
                bs_v.at[pl.ds(0, k)], sem1)
            cp2 = pltpu.async_copy(
                td_hbm.at[idxd_v.at[pl.ds(off, k)]],
                bd_v.at[pl.ds(0, k)], sem2)
            cp1.wait()
            cp2.wait()
            sl = pl.ds(base + off, k)
            pltpu.sync_copy(bs_v.at[pl.ds(0, k)], gs_hbm.at[sl])
            pltpu.sync_copy(bd_v.at[pl.ds(0, k)], gd_hbm.at[sl])

        def body(i, carry):
            chunk(i * KG, KG)
            return carry

        lax.fori_loop(0, NFULL, body, 0)
        chunk(NFULL * KG, KT)

    return gather


_gathers = [_make_gather(k) for k in range(NSL)]


BE = 3200
NBLK = ES // BE


def _edges_body(attr_ref, gs_ref, gd_ref,
                ee_w0_ref, ee_b0_ref, ee_w1_ref, ee_b1_ref, ee_w2t_ref,
                ee_b2_ref, ee_g_ref, ee_bb_ref,
                we_ref, pe_w1_ref, pe_b1_ref, pe_w2t_ref,
                pe_b2_ref, pe_g_ref, pe_bb_ref,
                e2w_ref):
    def silu(v):
        return v * jax.nn.sigmoid(v)

    def head_ln(hmat, w2t_ref, b2_ref, g_ref, bb_ref):
        qt = lax.dot_general(
            w2t_ref[...], hmat, (((1,), (1,)), ((), ())),
            preferred_element_type=jnp.float32) + b2_ref[...]
        m = (qt[0:1, :] + qt[1:2, :]) * 0.5
        d0 = qt[0:1, :] - m
        r = lax.rsqrt(d0 * d0 + 1e-5)
        g = g_ref[...]
        bb = bb_ref[...]
        r0 = d0 * r * g[0:1, 0:1] + bb[0:1, 0:1]
        r1 = -d0 * r * g[1:2, 0:1] + bb[1:2, 0:1]
        return r0, r1

    def outer(rowvec, wrow_ref):
        return lax.dot_general(
            rowvec, wrow_ref, (((0,), (0,)), ((), ())),
            preferred_element_type=jnp.float32)

    a = attr_ref[...]
    w0 = ee_w0_ref[...]
    h = silu(a[:, 0:1] * w0[0:1, :] + a[:, 1:2] * w0[1:2, :] + ee_b0_ref[...])
    h = silu(jnp.dot(h, ee_w1_ref[...], preferred_element_type=jnp.float32)
             + ee_b1_ref[...])
    e0r, e1r = head_ln(h, ee_w2t_ref, ee_b2_ref, ee_g_ref, ee_bb_ref)

    we = we_ref[...]
    h2 = silu(outer(e0r, we[0:1, :]) + outer(e1r, we[1:2, :])
              + _unpack_bf16(gs_ref[...]) + _unpack_bf16(gd_ref[...]))
    h2 = silu(jnp.dot(h2, pe_w1_ref[...], preferred_element_type=jnp.float32)
              + pe_b1_ref[...])
    l0, l1 = head_ln(h2, pe_w2t_ref, pe_b2_ref, pe_g_ref, pe_bb_ref)
    e20 = l0 + e0r
    e21 = l1 + e1r
    e2w_ref[...] = jnp.concatenate(
        [e20, e21, jnp.zeros((6, BE), jnp.float32)], axis=0)


def _w_spec(shape):
    return pl.BlockSpec(shape, lambda i: tuple(0 for _ in shape))


def _make_edges(slice_k):
    off = slice_k * NBLK
    return pl.pallas_call(
        _edges_body,
        grid=(NBLK,),
        in_specs=[
            pl.BlockSpec((BE, 2), lambda i: (off + i, 0)),
            pl.BlockSpec((BE, HP), lambda i: (i, 0)),
            pl.BlockSpec((BE, HP), lambda i: (i, 0)),
            _w_spec((2, H)), _w_spec((1, H)), _w_spec((H, H)),
            _w_spec((1, H)),
            _w_spec((2, H)), _w_spec((2, 1)), _w_spec((2, 1)),
            _w_spec((2, 1)),
            _w_spec((2, H)), _w_spec((H, H)), _w_spec((1, H)),
            _w_spec((2, H)), _w_spec((2, 1)), _w_spec((2, 1)),
            _w_spec((2, 1)),
        ],
        out_specs=pl.BlockSpec((8, BE), lambda i: (0, i)),
        out_shape=jax.ShapeDtypeStruct((8, ES), jnp.float32),
    )


_edges_calls = [_make_edges(k) for k in range(NSL)]


TSC = 10
TSO = 16
CH = ES // TSC
NOFF = 10240
NPAD = 2 * NOFF


def _make_scatter(slice_k):
    sbase = slice_k * ES

    @functools.partial(
        pl.kernel,
        out_type=jax.ShapeDtypeStruct((TSO, NPAD), jnp.float32),
        mesh=plsc.VectorSubcoreMesh(core_axis_name="c", subcore_axis_name="s"),
        scratch_types=[
            pltpu.VMEM((CH,), jnp.int32),
            pltpu.VMEM((8, CH), jnp.float32),
            pltpu.VMEM((NPAD,), jnp.float32),
        ],
        compiler_params=pltpu.CompilerParams(needs_layout_passes=False),
    )
    def scatter(ei_hbm, e2w_hbm, out_hbm, dst_v, v_v, acc_v):
        wid = lax.axis_index("s") * NC + lax.axis_index("c")

        @pl.when(wid < TSO)
        def _():
            def zero(i, carry):
                acc_v[pl.ds(i * 16, 16)] = jnp.zeros((16,), jnp.float32)
                return carry

            lax.fori_loop(0, NPAD // 16, zero, 0)

            @pl.when(wid < TSC)
            def _():
                b = wid * CH
                pltpu.sync_copy(ei_hbm.at[pl.ds(E + sbase + b, CH)], dst_v)
                pltpu.sync_copy(
                    e2w_hbm.at[pl.ds(0, 8), pl.ds(b, CH)], v_v)

                def inner(j, c2):
                    sl = pl.ds(j * 16, 16)
                    dv = dst_v[sl]
                    plsc.addupdate_scatter(acc_v, [dv], v_v[0, sl])
                    plsc.addupdate_scatter(acc_v, [dv + NOFF], v_v[1, sl])
                    return c2

                lax.fori_loop(0, CH // 16, inner, 0)

            pltpu.sync_copy(acc_v, out_hbm.at[wid])

    return scatter


_scatters = [_make_scatter(k) for k in range(NSL)]


_CHUNKS = [(0, 2048), (2048, 2048), (4096, 2048), (6144, 2048),
           (8192, 1808)]


def _nodes_body(x_ref, p0, p1, p2, p3, p4, st_ref,
                wx_ref, wa_ref, pn_b0_ref, pn_w1_ref, pn_b1_ref,
                pn_w2_ref, pn_b2_ref, pn_g_ref, pn_bb_ref,
                nd_w0_ref, nd_b0_ref, nd_w1_ref, nd_b1_ref,
                nd_w2_ref, nd_b2_ref, out_ref):
    def silu(v):
        return v * jax.nn.sigmoid(v)

    def outer(rowvec, wrow):
        return lax.dot_general(
            rowvec, wrow, (((0,), (0,)), ((), ())),
            preferred_element_type=jnp.float32)

    stot = (p0[...] + p1[...] + p2[...] + p3[...]
            + p4[...])
    agg = jnp.sum(stot, axis=0, keepdims=True)
    wa = wa_ref[...]

    for r0, rn in _CHUNKS:
        rows = pl.ds(r0, rn)
        agg0 = agg[:, r0:r0 + rn]
        agg1 = agg[:, NOFF + r0:NOFF + r0 + rn]
        x = x_ref[rows, :]
        h = silu(jnp.dot(x, wx_ref[...],
                         preferred_element_type=jnp.float32)
                 + outer(agg0, wa[0:1, :]) + outer(agg1, wa[1:2, :])
                 + pn_b0_ref[...])
        h = silu(jnp.dot(h, pn_w1_ref[...],
                         preferred_element_type=jnp.float32)
                 + pn_b1_ref[...])
        xp = (jnp.dot(h, pn_w2_ref[...],
                      preferred_element_type=jnp.float32)
              + pn_b2_ref[...])
        mu = jnp.mean(xp, axis=-1, keepdims=True)
        ctr = xp - mu
        va = jnp.mean(ctr * ctr, axis=-1, keepdims=True)
        x2 = ctr * lax.rsqrt(va + 1e-5) * pn_g_ref[...] + pn_bb_ref[...] + x
        dd = silu(jnp.dot(x2, nd_w0_ref[...],
                          preferred_element_type=jnp.float32)
                  + nd_b0_ref[...])
        dd = silu(jnp.dot(dd, nd_w1_ref[...],
                          preferred_element_type=jnp.float32)
                  + nd_b1_ref[...])
        out_ref[rows, :] = (jnp.dot(dd, nd_w2_ref[...],
                                    preferred_element_type=jnp.float32)
                            + nd_b2_ref[...] + st_ref[rows, :])


_nodes = pl.pallas_call(
    _nodes_body,
    out_shape=jax.ShapeDtypeStruct((N, D_OUT), jnp.float32),
)


def kernel(processor_features, start_features, edge_attr, edge_index,
           ee_w0, ee_b0, ee_w1, ee_b1, ee_w2, ee_b2, ee_ln_g, ee_ln_b,
           pe_w0, pe_b0, pe_w1, pe_b1, pe_w2, pe_b2, pe_ln_g, pe_ln_b,
           pn_w0, pn_b0, pn_w1, pn_b1, pn_w2, pn_b2, pn_ln_g, pn_ln_b,
           nd_w0, nd_b0, nd_w1, nd_b1, nd_w2, nd_b2):
    x = processor_features
    we = pe_w0[0:2]
    ws = pe_w0[2:2 + D_IN]
    wd = pe_w0[2 + D_IN:2 + 2 * D_IN]

    ts_p, td_p = _tables(x, ws, wd, pe_b0.reshape(1, H))

    ew_args = (
        ee_w0, ee_b0.reshape(1, H), ee_w1, ee_b1.reshape(1, H),
        ee_w2.T, ee_b2.reshape(2, 1), ee_ln_g.reshape(2, 1),
        ee_ln_b.reshape(2, 1),
        we, pe_w1, pe_b1.reshape(1, H), pe_w2.T, pe_b2.reshape(2, 1),
        pe_ln_g.reshape(2, 1), pe_ln_b.reshape(2, 1))

    ei_flat = edge_index.reshape(2 * E)
    parts = []
    for k in range(NSL):
        gs_k, gd_k = _gathers[k](ei_flat, ts_p, td_p)
        e2w_k = _edges_calls[k](edge_attr, gs_k, gd_k, *ew_args)
        parts.append(_scatters[k](ei_flat, e2w_k))

    out = _nodes(
        x, *parts, start_features,
        pn_w0[:D_IN], pn_w0[D_IN:], pn_b0.reshape(1, H),
        pn_w1, pn_b1.reshape(1, H), pn_w2, pn_b2.reshape(1, H),
        pn_ln_g.reshape(1, H), pn_ln_b.reshape(1, H),
        nd_w0, nd_b0.reshape(1, H_DEC), nd_w1, nd_b1.reshape(1, H_DEC),
        nd_w2, nd_b2.reshape(1, D_OUT))
    return out
